# Initial kernel scaffold; baseline (speedup 1.0000x reference)
import jax
import jax.numpy as jnp
from jax.experimental import pallas as pl


def kernel(x, edge_index, W1, b1, gamma, beta, W2, b2, eps):
    raise NotImplementedError("write your pallas kernel here")


# speedup vs baseline: 3.1134x; 3.1134x over previous
import functools

import jax
import jax.numpy as jnp
from jax import lax
from jax.experimental import pallas as pl
from jax.experimental.pallas import tpu as pltpu
from jax.experimental.pallas import tpu_sc as plsc

N_NODES = 10000
D = 128
E = 320000
NC = 2
NS = 16
NW = NC * NS
C = 128
K = 80
EPT = K * C
E_PAD = NW * EPT
N_PAD = 10240
RPT = N_PAD // NS

_mesh = plsc.VectorSubcoreMesh(core_axis_name="c", subcore_axis_name="s")


@functools.partial(
    pl.kernel,
    out_type=jax.ShapeDtypeStruct((NC, N_PAD, D), jnp.float32),
    mesh=_mesh,
    scratch_types=[
        pltpu.VMEM((K, C), jnp.int32),
        pltpu.VMEM((K, C), jnp.int32),
        pltpu.VMEM((C, D), jnp.float32),
        pltpu.VMEM_SHARED((N_PAD, D), jnp.float32),
        pltpu.SemaphoreType.DMA,
    ],
)
def _segsum_sc(src_hbm, dst_hbm, x_hbm, out_hbm, src_v, dst_v, rows_v, acc_sh, sem):
    cid = lax.axis_index("c")
    sid = lax.axis_index("s")
    wid = sid * NC + cid

    pltpu.sync_copy(src_hbm.at[wid], src_v)
    pltpu.sync_copy(dst_hbm.at[wid], dst_v)

    zeros16 = jnp.zeros((16,), jnp.float32)

    @pl.loop(0, C)
    def _zero_rows(r):
        for j in range(D // 16):
            rows_v[r, pl.ds(j * 16, 16)] = zeros16

    base = sid * RPT
    for t in range(RPT // C):
        pltpu.sync_copy(rows_v, acc_sh.at[pl.ds(base + t * C, C)])

    plsc.subcore_barrier()

    @pl.loop(0, K)
    def _edge_chunk(i):
        pltpu.async_copy(x_hbm.at[src_v.at[i]], rows_v, sem).wait()
        pltpu.sync_copy(rows_v, acc_sh.at[dst_v.at[i]], add=True)

    plsc.subcore_barrier()

    pltpu.sync_copy(acc_sh.at[pl.ds(base, RPT)], out_hbm.at[cid, pl.ds(base, RPT)])


def _mlp_body(x_ref, p_ref, w1t_ref, b1_ref, g_ref, be_ref, w2t_ref, b2_ref,
              eps_ref, o_ref):
    h = (x_ref[...] * (1.0 + eps_ref[0, 0])
         + p_ref[0, :N_NODES, :] + p_ref[1, :N_NODES, :])
    z = jnp.dot(h, w1t_ref[...], preferred_element_type=jnp.float32) + b1_ref[...]
    mean = jnp.mean(z, axis=0, keepdims=True)
    zc = z - mean
    var = jnp.mean(zc * zc, axis=0, keepdims=True)
    y = zc * lax.rsqrt(var + 1e-5) * g_ref[...] + be_ref[...]
    y = jnp.maximum(y, 0.0)
    o_ref[...] = jnp.dot(y, w2t_ref[...], preferred_element_type=jnp.float32) + b2_ref[...]


def _mlp_tc(x, partials, W1t, b1, gamma, beta, W2t, b2, eps):
    return pl.pallas_call(
        _mlp_body,
        out_shape=jax.ShapeDtypeStruct((N_NODES, D), jnp.float32),
    )(x, partials, W1t, b1.reshape(1, D), gamma.reshape(1, D),
      beta.reshape(1, D), W2t, b2.reshape(1, D), eps.reshape(1, 1))


def kernel(x, edge_index, W1, b1, gamma, beta, W2, b2, eps):
    pad = E_PAD - E
    src = jnp.concatenate(
        [edge_index[0].astype(jnp.int32), jnp.zeros((pad,), jnp.int32)]
    ).reshape(NW, K, C)
    junk = N_NODES + (jnp.arange(pad, dtype=jnp.int32) % (N_PAD - N_NODES))
    dst = jnp.concatenate(
        [edge_index[1].astype(jnp.int32), junk]
    ).reshape(NW, K, C)
    partials = _segsum_sc(src, dst, x)
    return _mlp_tc(x, partials, W1.T, b1, gamma, beta, W2.T, b2, eps)


# speedup vs baseline: 3.4450x; 1.1065x over previous
import functools

import jax
import jax.numpy as jnp
from jax import lax
from jax.experimental import pallas as pl
from jax.experimental.pallas import tpu as pltpu
from jax.experimental.pallas import tpu_sc as plsc

N_NODES = 10000
D = 128
E = 320000
NC = 2
NS = 16
NW = NC * NS
C = 128
K = 80
EPT = K * C
E_PAD = NW * EPT
N_PAD = 10240
RPT = N_PAD // NS
NBUF = 2

_mesh = plsc.VectorSubcoreMesh(core_axis_name="c", subcore_axis_name="s")


@functools.partial(
    pl.kernel,
    out_type=jax.ShapeDtypeStruct((NC, N_PAD, D), jnp.float32),
    mesh=_mesh,
    scratch_types=[
        pltpu.VMEM((K, C), jnp.int32),
        pltpu.VMEM((NBUF, C), jnp.int32),
        pltpu.VMEM((NBUF, C, D), jnp.float32),
        pltpu.VMEM_SHARED((N_PAD, D), jnp.float32),
        pltpu.SemaphoreType.DMA,
        pltpu.SemaphoreType.DMA,
        pltpu.SemaphoreType.DMA,
        pltpu.SemaphoreType.DMA,
    ],
)
def _segsum_sc(src_hbm, dst_hbm, x_hbm, out_hbm, src_v, dst_r, rows_v, acc_sh,
               gsem0, gsem1, dsem0, dsem1):
    gsems = (gsem0, gsem1)
    dsems = (dsem0, dsem1)
    cid = lax.axis_index("c")
    sid = lax.axis_index("s")
    wid = sid * NC + cid

    pltpu.sync_copy(src_hbm.at[wid], src_v)

    zeros16 = jnp.zeros((16,), jnp.float32)

    @pl.loop(0, C)
    def _zero_rows(r):
        for j in range(D // 16):
            rows_v[0, r, pl.ds(j * 16, 16)] = zeros16

    base = sid * RPT
    for t in range(RPT // C):
        pltpu.sync_copy(rows_v.at[0], acc_sh.at[pl.ds(base + t * C, C)])

    plsc.subcore_barrier()

    def _start_gather(chunk, b):
        pltpu.async_copy(x_hbm.at[src_v.at[chunk]], rows_v.at[b], gsems[b])

    def _wait_gather(b):
        pltpu.make_async_copy(x_hbm.at[src_v.at[0]], rows_v.at[b], gsems[b]).wait()

    def _start_dst(chunk, b):
        pltpu.async_copy(dst_hbm.at[wid, chunk], dst_r.at[b], dsems[b])

    def _wait_dst(b):
        pltpu.make_async_copy(dst_hbm.at[wid, 0], dst_r.at[b], dsems[b]).wait()

    def _scatter(b):
        pltpu.sync_copy(rows_v.at[b], acc_sh.at[dst_r.at[b]], add=True)

    for b in range(NBUF):
        _start_gather(b, b)
        _start_dst(b, b)

    @pl.loop(0, K - NBUF, step=NBUF)
    def _edge_group(i0):
        for b in range(NBUF):
            _wait_gather(b)
            _wait_dst(b)
            _scatter(b)
            _start_gather(i0 + NBUF + b, b)
            _start_dst(i0 + NBUF + b, b)

    for b in range(NBUF):
        _wait_gather(b)
        _wait_dst(b)
        _scatter(b)

    plsc.subcore_barrier()

    pltpu.sync_copy(acc_sh.at[pl.ds(base, RPT)], out_hbm.at[cid, pl.ds(base, RPT)])


def _mlp_body(x_ref, p_ref, w1t_ref, b1_ref, g_ref, be_ref, w2t_ref, b2_ref,
              eps_ref, o_ref):
    h = (x_ref[...] * (1.0 + eps_ref[0, 0])
         + p_ref[0, :N_NODES, :] + p_ref[1, :N_NODES, :])
    z = jnp.dot(h, w1t_ref[...], preferred_element_type=jnp.float32) + b1_ref[...]
    mean = jnp.mean(z, axis=0, keepdims=True)
    zc = z - mean
    var = jnp.mean(zc * zc, axis=0, keepdims=True)
    y = zc * lax.rsqrt(var + 1e-5) * g_ref[...] + be_ref[...]
    y = jnp.maximum(y, 0.0)
    o_ref[...] = jnp.dot(y, w2t_ref[...], preferred_element_type=jnp.float32) + b2_ref[...]


def _mlp_tc(x, partials, W1t, b1, gamma, beta, W2t, b2, eps):
    return pl.pallas_call(
        _mlp_body,
        out_shape=jax.ShapeDtypeStruct((N_NODES, D), jnp.float32),
    )(x, partials, W1t, b1.reshape(1, D), gamma.reshape(1, D),
      beta.reshape(1, D), W2t, b2.reshape(1, D), eps.reshape(1, 1))


def kernel(x, edge_index, W1, b1, gamma, beta, W2, b2, eps):
    pad = E_PAD - E
    src = jnp.concatenate(
        [edge_index[0].astype(jnp.int32), jnp.zeros((pad,), jnp.int32)]
    ).reshape(NW, K, C)
    junk = N_NODES + (jnp.arange(pad, dtype=jnp.int32) % (N_PAD - N_NODES))
    dst = jnp.concatenate(
        [edge_index[1].astype(jnp.int32), junk]
    ).reshape(NW, K, C)
    partials = _segsum_sc(src, dst, x)
    return _mlp_tc(x, partials, W1.T, b1, gamma, beta, W2.T, b2, eps)
